# baseline (speedup 1.0000x reference)
import functools

import jax
import jax.numpy as jnp
from jax import lax
from jax.experimental import pallas as pl
from jax.experimental.pallas import tpu as pltpu
from jax.experimental.pallas import tpu_sc as plsc

NC, NS = 2, 16
NW = NC * NS
CHUNK = 128
BATCH, HIST, DIM = 16384, 50, 64
TOTAL = BATCH * HIST
NCHUNK_ALL = TOTAL // CHUNK
PER_W = NCHUNK_ALL // NW

_mesh = plsc.VectorSubcoreMesh(core_axis_name="c", subcore_axis_name="s")


@functools.partial(
    pl.kernel,
    out_type=jax.ShapeDtypeStruct((TOTAL, DIM), jnp.float32),
    mesh=_mesh,
    scratch_types=[
        pltpu.VMEM((PER_W, CHUNK), jnp.int32),
        pltpu.VMEM((4, CHUNK, DIM), jnp.float32),
        pltpu.SemaphoreType.DMA((4,)),
        pltpu.SemaphoreType.DMA((4,)),
    ],
    compiler_params=pltpu.CompilerParams(use_tc_tiling_on_sc=False),
)
def _gather(idx_hbm, table_hbm, out_hbm, idx_v, rows_v, sem_g, sem_w):
    wid = lax.axis_index("s") * NC + lax.axis_index("c")
    pltpu.sync_copy(idx_hbm.at[wid], idx_v)
    base_c = wid * PER_W

    def gather_desc(t):
        b = lax.rem(t, 4)
        return pltpu.make_async_copy(
            table_hbm.at[idx_v.at[t]],
            rows_v.at[b],
            sem_g.at[b],
        )

    def write_desc(t):
        b = lax.rem(t, 4)
        return pltpu.make_async_copy(
            rows_v.at[b],
            out_hbm.at[pl.ds((base_c + t) * CHUNK, CHUNK)],
            sem_w.at[b],
        )

    gather_desc(0).start()
    gather_desc(1).start()

    def body(t):
        gather_desc(t).wait()
        write_desc(t).start()

        @pl.when(t + 2 < PER_W)
        def _():
            @pl.when(t >= 2)
            def _():
                write_desc(t - 2).wait()

            gather_desc(t + 2).start()

    pl.loop(0, PER_W)(body)
    write_desc(PER_W - 2).wait()
    write_desc(PER_W - 1).wait()


NROWS = 1000000
TBLK = 1024
NTBLK = (NROWS + TBLK - 1) // TBLK
TPAD = NTBLK * TBLK


def _relayout_body(wt_ref, o_ref):
    b = wt_ref[:].T.reshape(TBLK // 2, 2, DIM)
    o_ref[:, 0:DIM] = b[:, 0]
    o_ref[:, DIM : 2 * DIM] = b[:, 1]


def _relayout(Wt):
    return pl.pallas_call(
        _relayout_body,
        grid=(NTBLK,),
        in_specs=[pl.BlockSpec((DIM, TBLK), lambda i: (0, i))],
        out_specs=pl.BlockSpec((TBLK // 2, 2 * DIM), lambda i: (i, 0)),
        out_shape=jax.ShapeDtypeStruct((TPAD // 2, 2 * DIM), jnp.float32),
    )(Wt)


def kernel(topic_ids, W):
    table = _relayout(W.T).reshape(TPAD, DIM)
    idx = topic_ids.reshape(NW, PER_W, CHUNK)
    q = _gather(idx, table)
    return q.reshape(BATCH, HIST, DIM), 0


# speedup vs baseline: 1.1084x; 1.1084x over previous
import functools

import jax
import jax.numpy as jnp
from jax import lax
from jax.experimental import pallas as pl
from jax.experimental.pallas import tpu as pltpu
from jax.experimental.pallas import tpu_sc as plsc

NC, NS = 2, 16
NW = NC * NS
CHUNK = 128
BATCH, HIST, DIM = 16384, 50, 64
TOTAL = BATCH * HIST
NCHUNK_ALL = TOTAL // CHUNK
PER_W = NCHUNK_ALL // NW

_mesh = plsc.VectorSubcoreMesh(core_axis_name="c", subcore_axis_name="s")


@functools.partial(
    pl.kernel,
    out_type=jax.ShapeDtypeStruct((TOTAL, DIM), jnp.float32),
    mesh=_mesh,
    scratch_types=[
        pltpu.VMEM((PER_W, CHUNK), jnp.int32),
        pltpu.VMEM((4, CHUNK, DIM), jnp.float32),
        pltpu.SemaphoreType.DMA((4,)),
        pltpu.SemaphoreType.DMA((4,)),
    ],
    compiler_params=pltpu.CompilerParams(use_tc_tiling_on_sc=False),
)
def _gather(idx_hbm, table_hbm, out_hbm, idx_v, rows_v, sem_g, sem_w):
    wid = lax.axis_index("s") * NC + lax.axis_index("c")
    pltpu.sync_copy(idx_hbm.at[wid], idx_v)
    base_c = wid * PER_W

    def gather_desc(t):
        b = lax.rem(t, 4)
        return pltpu.make_async_copy(
            table_hbm.at[idx_v.at[t]],
            rows_v.at[b],
            sem_g.at[b],
        )

    def write_desc(t):
        b = lax.rem(t, 4)
        return pltpu.make_async_copy(
            rows_v.at[b],
            out_hbm.at[pl.ds((base_c + t) * CHUNK, CHUNK)],
            sem_w.at[b],
        )

    gather_desc(0).start()
    gather_desc(1).start()

    def body(t):
        gather_desc(t).wait()
        write_desc(t).start()

        @pl.when(t + 2 < PER_W)
        def _():
            @pl.when(t >= 2)
            def _():
                write_desc(t - 2).wait()

            gather_desc(t + 2).start()

    pl.loop(0, PER_W)(body)
    write_desc(PER_W - 2).wait()
    write_desc(PER_W - 1).wait()


NROWS = 1000000
TBLK = 1024
NTBLK = (NROWS + TBLK - 1) // TBLK
TPAD = NTBLK * TBLK


def _relayout_body(wt_ref, o_ref):
    z = jnp.concatenate(
        [wt_ref[:, 0 : TBLK // 2], wt_ref[:, TBLK // 2 : TBLK]], axis=0
    )
    o_ref[:] = z.T


def _relayout(Wt):
    return pl.pallas_call(
        _relayout_body,
        grid=(NTBLK,),
        in_specs=[pl.BlockSpec((DIM, TBLK), lambda i: (0, i))],
        out_specs=pl.BlockSpec((TBLK // 2, 2 * DIM), lambda i: (i, 0)),
        out_shape=jax.ShapeDtypeStruct((TPAD // 2, 2 * DIM), jnp.float32),
    )(Wt)


def kernel(topic_ids, W):
    table = _relayout(W.T).reshape(TPAD, DIM)
    ids = (topic_ids & ~1023) | ((topic_ids & 511) << 1) | ((topic_ids >> 9) & 1)
    idx = ids.reshape(NW, PER_W, CHUNK)
    q = _gather(idx, table)
    return q.reshape(BATCH, HIST, DIM), 0


# speedup vs baseline: 1.6595x; 1.4973x over previous
import functools

import jax
import jax.numpy as jnp
from jax import lax
from jax.experimental import pallas as pl
from jax.experimental.pallas import tpu as pltpu
from jax.experimental.pallas import tpu_sc as plsc

NC, NS = 2, 16
NW = NC * NS
CHUNK = 128
BATCH, HIST, DIM = 16384, 50, 64
TOTAL = BATCH * HIST
NCHUNK_ALL = TOTAL // CHUNK
PER_W = NCHUNK_ALL // NW

_mesh = plsc.VectorSubcoreMesh(core_axis_name="c", subcore_axis_name="s")


@functools.partial(
    pl.kernel,
    out_type=jax.ShapeDtypeStruct((TOTAL, DIM), jnp.float32),
    mesh=_mesh,
    scratch_types=[
        pltpu.VMEM((PER_W, CHUNK), jnp.int32),
        pltpu.VMEM((4, CHUNK, DIM), jnp.float32),
        pltpu.SemaphoreType.DMA((4,)),
        pltpu.SemaphoreType.DMA((4,)),
    ],
    compiler_params=pltpu.CompilerParams(use_tc_tiling_on_sc=False),
)
def _gather(idx_hbm, table_hbm, out_hbm, idx_v, rows_v, sem_g, sem_w):
    wid = lax.axis_index("s") * NC + lax.axis_index("c")
    pltpu.sync_copy(idx_hbm.at[wid], idx_v)
    base_c = wid * PER_W

    def gather_desc(t):
        b = lax.rem(t, 4)
        return pltpu.make_async_copy(
            table_hbm.at[idx_v.at[t]],
            rows_v.at[b],
            sem_g.at[b],
        )

    def write_desc(t):
        b = lax.rem(t, 4)
        return pltpu.make_async_copy(
            rows_v.at[b],
            out_hbm.at[pl.ds((base_c + t) * CHUNK, CHUNK)],
            sem_w.at[b],
        )

    gather_desc(0).start()
    gather_desc(1).start()

    def body(t):
        gather_desc(t).wait()
        write_desc(t).start()

        @pl.when(t + 2 < PER_W)
        def _():
            @pl.when(t >= 2)
            def _():
                write_desc(t - 2).wait()

            gather_desc(t + 2).start()

    pl.loop(0, PER_W)(body)
    write_desc(PER_W - 2).wait()
    write_desc(PER_W - 1).wait()


NROWS = 1000000
TBLK = 8192
HALF = TBLK // 2
LOGH = HALF.bit_length() - 1
NTBLK = (NROWS + TBLK - 1) // TBLK
TPAD = NTBLK * TBLK


def _relayout_body(wt_ref, o_ref):
    z = jnp.concatenate([wt_ref[:, 0:HALF], wt_ref[:, HALF:TBLK]], axis=0)
    o_ref[:] = z.T


def _relayout(Wt):
    return pl.pallas_call(
        _relayout_body,
        grid=(NTBLK,),
        in_specs=[pl.BlockSpec((DIM, TBLK), lambda i: (0, i))],
        out_specs=pl.BlockSpec((HALF, 2 * DIM), lambda i: (i, 0)),
        out_shape=jax.ShapeDtypeStruct((TPAD // 2, 2 * DIM), jnp.float32),
    )(Wt)


def kernel(topic_ids, W):
    table = _relayout(W.T).reshape(TPAD, DIM)
    ids = (
        (topic_ids & ~(TBLK - 1))
        | ((topic_ids & (HALF - 1)) << 1)
        | ((topic_ids >> LOGH) & 1)
    )
    idx = ids.reshape(NW, PER_W, CHUNK)
    q = _gather(idx, table)
    return q.reshape(BATCH, HIST, DIM), 0


# speedup vs baseline: 2.2871x; 1.3782x over previous
import functools

import jax
import jax.numpy as jnp
from jax import lax
from jax.experimental import pallas as pl
from jax.experimental.pallas import tpu as pltpu
from jax.experimental.pallas import tpu_sc as plsc

NC, NS = 2, 16
NW = NC * NS
CHUNK = 128
BATCH, HIST, DIM = 16384, 50, 64
TOTAL = BATCH * HIST
NCHUNK_ALL = TOTAL // CHUNK
PER_W = NCHUNK_ALL // NW

_mesh = plsc.VectorSubcoreMesh(core_axis_name="c", subcore_axis_name="s")


@functools.partial(
    pl.kernel,
    out_type=jax.ShapeDtypeStruct((NCHUNK_ALL, CHUNK, DIM), jnp.float32),
    mesh=_mesh,
    scratch_types=[
        pltpu.VMEM((PER_W, CHUNK), jnp.int32),
        pltpu.VMEM((4, CHUNK, DIM), jnp.float32),
        pltpu.SemaphoreType.DMA((4,)),
        pltpu.SemaphoreType.DMA((4,)),
    ],
    compiler_params=pltpu.CompilerParams(use_tc_tiling_on_sc=False),
)
def _gather(idx_hbm, table_hbm, out_hbm, idx_v, rows_v, sem_g, sem_w):
    wid = lax.axis_index("s") * NC + lax.axis_index("c")
    pltpu.sync_copy(idx_hbm.at[wid], idx_v)
    base_c = wid * PER_W

    def gather_desc(t):
        b = lax.rem(t, 4)
        return pltpu.make_async_copy(
            table_hbm.at[idx_v.at[t]],
            rows_v.at[b],
            sem_g.at[b],
        )

    def write_desc(t):
        b = lax.rem(t, 4)
        return pltpu.make_async_copy(
            rows_v.at[b],
            out_hbm.at[base_c + t],
            sem_w.at[b],
        )

    gather_desc(0).start()
    gather_desc(1).start()

    def body(t):
        gather_desc(t).wait()
        write_desc(t).start()

        @pl.when(t + 2 < PER_W)
        def _():
            @pl.when(t >= 2)
            def _():
                write_desc(t - 2).wait()

            gather_desc(t + 2).start()

    pl.loop(0, PER_W)(body)
    write_desc(PER_W - 2).wait()
    write_desc(PER_W - 1).wait()


NROWS = 1000000
TBLK = 8192
HALF = TBLK // 2
LOGH = HALF.bit_length() - 1
NTBLK = (NROWS + TBLK - 1) // TBLK
TPAD = NTBLK * TBLK


def _relayout_body(wt_ref, o_ref):
    z = jnp.concatenate([wt_ref[:, 0:HALF], wt_ref[:, HALF:TBLK]], axis=0)
    o_ref[:] = z.T


def _relayout(Wt):
    return pl.pallas_call(
        _relayout_body,
        grid=(NTBLK,),
        in_specs=[pl.BlockSpec((DIM, TBLK), lambda i: (0, i))],
        out_specs=pl.BlockSpec((HALF, 2 * DIM), lambda i: (i, 0)),
        out_shape=jax.ShapeDtypeStruct((TPAD // 2, 2 * DIM), jnp.float32),
    )(Wt)


NBLK = BATCH // CHUNK
GBT = 32


def _detrans_body(g_ref, o_ref):
    b = g_ref[0]
    s = jnp.concatenate([b[:, :, 0:DIM], b[:, :, DIM : 2 * DIM]], axis=1)
    st = s.transpose(0, 2, 1)
    q = st.reshape(GBT, 8, 8, CHUNK)
    o_ref[0] = q.transpose(1, 0, 2, 3)


def _detrans(Gv):
    return pl.pallas_call(
        _detrans_body,
        grid=(HIST, NBLK // GBT),
        in_specs=[pl.BlockSpec((1, GBT, DIM, CHUNK), lambda h, g: (h, g, 0, 0))],
        out_specs=pl.BlockSpec(
            (1, 8, GBT, 8, CHUNK), lambda h, g: (h, 0, g, 0, 0)
        ),
        out_shape=jax.ShapeDtypeStruct((HIST, 8, NBLK, 8, CHUNK), jnp.float32),
    )(Gv)


_PERM = tuple(k // 2 if k % 2 == 0 else 64 + k // 2 for k in range(CHUNK))


def kernel(topic_ids, W):
    table = _relayout(W.T).reshape(TPAD, DIM)
    ids = (
        (topic_ids & ~(TBLK - 1))
        | ((topic_ids & (HALF - 1)) << 1)
        | ((topic_ids >> LOGH) & 1)
    )
    perm = jnp.asarray(_PERM, jnp.int32)
    idx = ids.T.reshape(HIST, NBLK, CHUNK)[:, :, perm].reshape(NW, PER_W, CHUNK)
    q = _gather(idx, table)
    o5 = _detrans(q.reshape(HIST, NBLK, DIM, CHUNK))
    out = o5.transpose(2, 4, 0, 1, 3).reshape(BATCH, HIST, DIM)
    return out, 0


# speedup vs baseline: 2.3929x; 1.0463x over previous
import functools

import jax
import jax.numpy as jnp
from jax import lax
from jax.experimental import pallas as pl
from jax.experimental.pallas import tpu as pltpu
from jax.experimental.pallas import tpu_sc as plsc

NC, NS = 2, 16
NW = NC * NS
CHUNK = 128
BATCH, HIST, DIM = 16384, 50, 64
TOTAL = BATCH * HIST
NCHUNK_ALL = TOTAL // CHUNK
PER_W = NCHUNK_ALL // NW

_mesh = plsc.VectorSubcoreMesh(core_axis_name="c", subcore_axis_name="s")


@functools.partial(
    pl.kernel,
    out_type=jax.ShapeDtypeStruct((NCHUNK_ALL, CHUNK, DIM), jnp.float32),
    mesh=_mesh,
    scratch_types=[
        pltpu.VMEM((PER_W, CHUNK), jnp.int32),
        pltpu.VMEM((4, CHUNK, DIM), jnp.float32),
        pltpu.SemaphoreType.DMA((4,)),
        pltpu.SemaphoreType.DMA((4,)),
    ],
    compiler_params=pltpu.CompilerParams(use_tc_tiling_on_sc=False),
)
def _gather(idx_hbm, table_hbm, out_hbm, idx_v, rows_v, sem_g, sem_w):
    wid = lax.axis_index("s") * NC + lax.axis_index("c")
    pltpu.sync_copy(idx_hbm.at[wid], idx_v)
    base_c = wid * PER_W

    def gather_desc(t):
        b = lax.rem(t, 4)
        return pltpu.make_async_copy(
            table_hbm.at[idx_v.at[t]],
            rows_v.at[b],
            sem_g.at[b],
        )

    def write_desc(t):
        b = lax.rem(t, 4)
        return pltpu.make_async_copy(
            rows_v.at[b],
            out_hbm.at[base_c + t],
            sem_w.at[b],
        )

    gather_desc(0).start()
    gather_desc(1).start()

    def body(t):
        gather_desc(t).wait()
        write_desc(t).start()

        @pl.when(t + 2 < PER_W)
        def _():
            @pl.when(t >= 2)
            def _():
                write_desc(t - 2).wait()

            gather_desc(t + 2).start()

    pl.loop(0, PER_W)(body)
    write_desc(PER_W - 2).wait()
    write_desc(PER_W - 1).wait()


NROWS = 1000000
TBLK = 16384
HALF = TBLK // 2
LOGH = HALF.bit_length() - 1
NTBLK = (NROWS + TBLK - 1) // TBLK
TPAD = NTBLK * TBLK


def _relayout_body(wt_ref, o_ref):
    z = jnp.concatenate([wt_ref[:, 0:HALF], wt_ref[:, HALF:TBLK]], axis=0)
    o_ref[:] = z.T


def _relayout(Wt):
    return pl.pallas_call(
        _relayout_body,
        grid=(NTBLK,),
        in_specs=[pl.BlockSpec((DIM, TBLK), lambda i: (0, i))],
        out_specs=pl.BlockSpec((HALF, 2 * DIM), lambda i: (i, 0)),
        out_shape=jax.ShapeDtypeStruct((TPAD // 2, 2 * DIM), jnp.float32),
    )(Wt)


NBLK = BATCH // CHUNK
GBT = 32


def _detrans_body(g_ref, o_ref):
    b = g_ref[0]
    s = jnp.concatenate([b[:, :, 0:DIM], b[:, :, DIM : 2 * DIM]], axis=1)
    st = s.transpose(0, 2, 1)
    q = st.reshape(GBT, 8, 8, CHUNK)
    o_ref[0] = q.transpose(1, 0, 2, 3)


def _detrans(Gv):
    return pl.pallas_call(
        _detrans_body,
        grid=(HIST, NBLK // GBT),
        in_specs=[pl.BlockSpec((1, GBT, DIM, CHUNK), lambda h, g: (h, g, 0, 0))],
        out_specs=pl.BlockSpec(
            (1, 8, GBT, 8, CHUNK), lambda h, g: (h, 0, g, 0, 0)
        ),
        out_shape=jax.ShapeDtypeStruct((HIST, 8, NBLK, 8, CHUNK), jnp.float32),
    )(Gv)


_PERM = tuple(k // 2 if k % 2 == 0 else 64 + k // 2 for k in range(CHUNK))


def kernel(topic_ids, W):
    table = _relayout(W.T).reshape(TPAD, DIM)
    ids = (
        (topic_ids & ~(TBLK - 1))
        | ((topic_ids & (HALF - 1)) << 1)
        | ((topic_ids >> LOGH) & 1)
    )
    perm = jnp.asarray(_PERM, jnp.int32)
    idx = ids.T.reshape(HIST, NBLK, CHUNK)[:, :, perm].reshape(NW, PER_W, CHUNK)
    q = _gather(idx, table)
    o5 = _detrans(q.reshape(HIST, NBLK, DIM, CHUNK))
    out = o5.transpose(2, 4, 0, 1, 3).reshape(BATCH, HIST, DIM)
    return out, 0


# speedup vs baseline: 2.6564x; 1.1101x over previous
import functools

import jax
import jax.numpy as jnp
from jax import lax
from jax.experimental import pallas as pl
from jax.experimental.pallas import tpu as pltpu
from jax.experimental.pallas import tpu_sc as plsc

NC, NS = 2, 16
NW = NC * NS
CHUNK = 128
BATCH, HIST, DIM = 16384, 50, 64
TOTAL = BATCH * HIST
NCHUNK_ALL = TOTAL // CHUNK
PER_W = NCHUNK_ALL // NW

_mesh = plsc.VectorSubcoreMesh(core_axis_name="c", subcore_axis_name="s")


@functools.partial(
    pl.kernel,
    out_type=jax.ShapeDtypeStruct((NCHUNK_ALL, CHUNK, DIM), jnp.float32),
    mesh=_mesh,
    scratch_types=[
        pltpu.VMEM((PER_W, CHUNK), jnp.int32),
        pltpu.VMEM((4, CHUNK, DIM), jnp.float32),
        pltpu.SemaphoreType.DMA((4,)),
        pltpu.SemaphoreType.DMA((4,)),
    ],
    compiler_params=pltpu.CompilerParams(use_tc_tiling_on_sc=False),
)
def _gather(idx_hbm, table_hbm, out_hbm, idx_v, rows_v, sem_g, sem_w):
    wid = lax.axis_index("s") * NC + lax.axis_index("c")
    pltpu.sync_copy(idx_hbm.at[wid], idx_v)
    base_c = wid * PER_W

    def gather_desc(t):
        b = lax.rem(t, 4)
        return pltpu.make_async_copy(
            table_hbm.at[idx_v.at[t]],
            rows_v.at[b],
            sem_g.at[b],
        )

    def write_desc(t):
        b = lax.rem(t, 4)
        return pltpu.make_async_copy(
            rows_v.at[b],
            out_hbm.at[base_c + t],
            sem_w.at[b],
        )

    gather_desc(0).start()
    gather_desc(1).start()

    def body(t):
        gather_desc(t).wait()
        write_desc(t).start()

        @pl.when(t + 2 < PER_W)
        def _():
            @pl.when(t >= 2)
            def _():
                write_desc(t - 2).wait()

            gather_desc(t + 2).start()

    pl.loop(0, PER_W)(body)
    write_desc(PER_W - 2).wait()
    write_desc(PER_W - 1).wait()


NROWS = 1000000
TBLK = 32768
HALF = TBLK // 2
LOGH = HALF.bit_length() - 1
NTBLK = (NROWS + TBLK - 1) // TBLK
TPAD = NTBLK * TBLK


def _relayout_body(wt_ref, o_ref):
    z = jnp.concatenate([wt_ref[:, 0:HALF], wt_ref[:, HALF:TBLK]], axis=0)
    o_ref[:] = z.T


def _relayout(Wt):
    return pl.pallas_call(
        _relayout_body,
        grid=(NTBLK,),
        in_specs=[pl.BlockSpec((DIM, TBLK), lambda i: (0, i))],
        out_specs=pl.BlockSpec((HALF, 2 * DIM), lambda i: (i, 0)),
        out_shape=jax.ShapeDtypeStruct((TPAD // 2, 2 * DIM), jnp.float32),
    )(Wt)


NBLK = BATCH // CHUNK
GBT = 64


def _detrans_body(g_ref, o_ref):
    b = g_ref[0]
    s = jnp.concatenate([b[:, :, 0:DIM], b[:, :, DIM : 2 * DIM]], axis=1)
    st = s.transpose(0, 2, 1)
    q = st.reshape(GBT, 8, 8, CHUNK)
    o_ref[0] = q.transpose(1, 0, 2, 3)


def _detrans(Gv):
    return pl.pallas_call(
        _detrans_body,
        grid=(HIST, NBLK // GBT),
        in_specs=[pl.BlockSpec((1, GBT, DIM, CHUNK), lambda h, g: (h, g, 0, 0))],
        out_specs=pl.BlockSpec(
            (1, 8, GBT, 8, CHUNK), lambda h, g: (h, 0, g, 0, 0)
        ),
        out_shape=jax.ShapeDtypeStruct((HIST, 8, NBLK, 8, CHUNK), jnp.float32),
    )(Gv)


_PERM = tuple(k // 2 if k % 2 == 0 else 64 + k // 2 for k in range(CHUNK))


def kernel(topic_ids, W):
    table = _relayout(W.T).reshape(TPAD, DIM)
    ids = (
        (topic_ids & ~(TBLK - 1))
        | ((topic_ids & (HALF - 1)) << 1)
        | ((topic_ids >> LOGH) & 1)
    )
    perm = jnp.asarray(_PERM, jnp.int32)
    idx = ids.T.reshape(HIST, NBLK, CHUNK)[:, :, perm].reshape(NW, PER_W, CHUNK)
    q = _gather(idx, table)
    o5 = _detrans(q.reshape(HIST, NBLK, DIM, CHUNK))
    out = o5.transpose(2, 4, 0, 1, 3).reshape(BATCH, HIST, DIM)
    return out, 0


# speedup vs baseline: 2.7962x; 1.0526x over previous
import functools

import jax
import jax.numpy as jnp
from jax import lax
from jax.experimental import pallas as pl
from jax.experimental.pallas import tpu as pltpu
from jax.experimental.pallas import tpu_sc as plsc

NC, NS = 2, 16
NW = NC * NS
CHUNK = 128
BATCH, HIST, DIM = 16384, 50, 64
TOTAL = BATCH * HIST
NCHUNK_ALL = TOTAL // CHUNK
PER_W = NCHUNK_ALL // NW

_mesh = plsc.VectorSubcoreMesh(core_axis_name="c", subcore_axis_name="s")


@functools.partial(
    pl.kernel,
    out_type=jax.ShapeDtypeStruct((NCHUNK_ALL, CHUNK, DIM), jnp.float32),
    mesh=_mesh,
    scratch_types=[
        pltpu.VMEM((PER_W, CHUNK), jnp.int32),
        pltpu.VMEM((4, CHUNK, DIM), jnp.float32),
        pltpu.SemaphoreType.DMA((4,)),
        pltpu.SemaphoreType.DMA((4,)),
    ],
    compiler_params=pltpu.CompilerParams(use_tc_tiling_on_sc=False),
)
def _gather(idx_hbm, table_hbm, out_hbm, idx_v, rows_v, sem_g, sem_w):
    wid = lax.axis_index("s") * NC + lax.axis_index("c")
    pltpu.sync_copy(idx_hbm.at[wid], idx_v)
    base_c = wid * PER_W

    def gather_desc(t):
        b = lax.rem(t, 4)
        return pltpu.make_async_copy(
            table_hbm.at[idx_v.at[t]],
            rows_v.at[b],
            sem_g.at[b],
        )

    def write_desc(t):
        b = lax.rem(t, 4)
        return pltpu.make_async_copy(
            rows_v.at[b],
            out_hbm.at[base_c + t],
            sem_w.at[b],
        )

    gather_desc(0).start()
    gather_desc(1).start()

    def body(t):
        gather_desc(t).wait()
        write_desc(t).start()

        @pl.when(t + 2 < PER_W)
        def _():
            @pl.when(t >= 2)
            def _():
                write_desc(t - 2).wait()

            gather_desc(t + 2).start()

    pl.loop(0, PER_W)(body)
    write_desc(PER_W - 2).wait()
    write_desc(PER_W - 1).wait()


NROWS = 1000000
TBLK = 32768
HALF = TBLK // 2
LOGH = HALF.bit_length() - 1
NTBLK = (NROWS + TBLK - 1) // TBLK
TPAD = NTBLK * TBLK


def _relayout_body(wt_ref, o_ref):
    z = jnp.concatenate([wt_ref[:, 0:HALF], wt_ref[:, HALF:TBLK]], axis=0)
    o_ref[:] = z.T


def _relayout(Wt):
    return pl.pallas_call(
        _relayout_body,
        grid=(NTBLK,),
        in_specs=[pl.BlockSpec((DIM, TBLK), lambda i: (0, i))],
        out_specs=pl.BlockSpec((HALF, 2 * DIM), lambda i: (i, 0)),
        out_shape=jax.ShapeDtypeStruct((TPAD // 2, 2 * DIM), jnp.float32),
    )(Wt)


NBLK = BATCH // CHUNK
GBT = 128


def _detrans_body(g_ref, o_ref):
    b = g_ref[0]
    s = jnp.concatenate([b[:, :, 0:DIM], b[:, :, DIM : 2 * DIM]], axis=1)
    st = s.transpose(0, 2, 1)
    q = st.reshape(GBT, 8, 8, CHUNK)
    o_ref[0] = q.transpose(1, 0, 2, 3)


def _detrans(Gv):
    return pl.pallas_call(
        _detrans_body,
        grid=(HIST, NBLK // GBT),
        in_specs=[pl.BlockSpec((1, GBT, DIM, CHUNK), lambda h, g: (h, g, 0, 0))],
        out_specs=pl.BlockSpec(
            (1, 8, GBT, 8, CHUNK), lambda h, g: (h, 0, g, 0, 0)
        ),
        out_shape=jax.ShapeDtypeStruct((HIST, 8, NBLK, 8, CHUNK), jnp.float32),
    )(Gv)


_PERM = tuple(k // 2 if k % 2 == 0 else 64 + k // 2 for k in range(CHUNK))


def kernel(topic_ids, W):
    table = _relayout(W.T).reshape(TPAD, DIM)
    ids = (
        (topic_ids & ~(TBLK - 1))
        | ((topic_ids & (HALF - 1)) << 1)
        | ((topic_ids >> LOGH) & 1)
    )
    perm = jnp.asarray(_PERM, jnp.int32)
    idx = ids.T.reshape(HIST, NBLK, CHUNK)[:, :, perm].reshape(NW, PER_W, CHUNK)
    q = _gather(idx, table)
    o5 = _detrans(q.reshape(HIST, NBLK, DIM, CHUNK))
    out = o5.transpose(2, 4, 0, 1, 3).reshape(BATCH, HIST, DIM)
    return out, 0


# speedup vs baseline: 2.8538x; 1.0206x over previous
import functools

import jax
import jax.numpy as jnp
from jax import lax
from jax.experimental import pallas as pl
from jax.experimental.pallas import tpu as pltpu
from jax.experimental.pallas import tpu_sc as plsc

NC, NS = 2, 16
NW = NC * NS
CHUNK = 128
BATCH, HIST, DIM = 16384, 50, 64
TOTAL = BATCH * HIST
NCHUNK_ALL = TOTAL // CHUNK
PER_W = NCHUNK_ALL // NW

_mesh = plsc.VectorSubcoreMesh(core_axis_name="c", subcore_axis_name="s")


@functools.partial(
    pl.kernel,
    out_type=jax.ShapeDtypeStruct((NCHUNK_ALL, CHUNK, DIM), jnp.float32),
    mesh=_mesh,
    scratch_types=[
        pltpu.VMEM((PER_W, CHUNK), jnp.int32),
        pltpu.VMEM((4, CHUNK, DIM), jnp.float32),
        pltpu.SemaphoreType.DMA((4,)),
        pltpu.SemaphoreType.DMA((4,)),
    ],
    compiler_params=pltpu.CompilerParams(use_tc_tiling_on_sc=False),
)
def _gather(idx_hbm, table_hbm, out_hbm, idx_v, rows_v, sem_g, sem_w):
    wid = lax.axis_index("s") * NC + lax.axis_index("c")
    pltpu.sync_copy(idx_hbm.at[wid], idx_v)
    base_c = wid * PER_W

    def gather_desc(t):
        b = lax.rem(t, 4)
        return pltpu.make_async_copy(
            table_hbm.at[idx_v.at[t]],
            rows_v.at[b],
            sem_g.at[b],
        )

    def write_desc(t):
        b = lax.rem(t, 4)
        return pltpu.make_async_copy(
            rows_v.at[b],
            out_hbm.at[base_c + t],
            sem_w.at[b],
        )

    gather_desc(0).start()
    gather_desc(1).start()

    def body(t):
        gather_desc(t).wait()
        write_desc(t).start()

        @pl.when(t + 2 < PER_W)
        def _():
            @pl.when(t >= 2)
            def _():
                write_desc(t - 2).wait()

            gather_desc(t + 2).start()

    pl.loop(0, PER_W)(body)
    write_desc(PER_W - 2).wait()
    write_desc(PER_W - 1).wait()


NROWS = 1000000
TBLK = 32768
HALF = TBLK // 2
LOGH = HALF.bit_length() - 1
NTBLK = (NROWS + TBLK - 1) // TBLK
TPAD = NTBLK * TBLK


def _relayout_body(wt_ref, o_ref):
    z = jnp.concatenate([wt_ref[:, 0:HALF], wt_ref[:, HALF:TBLK]], axis=0)
    o_ref[:] = z.T


def _relayout(Wt):
    return pl.pallas_call(
        _relayout_body,
        grid=(NTBLK,),
        in_specs=[pl.BlockSpec((DIM, TBLK), lambda i: (0, i))],
        out_specs=pl.BlockSpec((HALF, 2 * DIM), lambda i: (i, 0)),
        out_shape=jax.ShapeDtypeStruct((TPAD // 2, 2 * DIM), jnp.float32),
    )(Wt)


NBLK = BATCH // CHUNK
GBT = 128


H2 = 2


def _detrans_body(g_ref, o_ref):
    b = g_ref[:]
    s = jnp.concatenate([b[:, :, :, 0:DIM], b[:, :, :, DIM : 2 * DIM]], axis=2)
    st = s.transpose(0, 1, 3, 2)
    q = st.reshape(H2, GBT, 8, 8, CHUNK)
    o_ref[:] = q.transpose(0, 2, 1, 3, 4)


def _detrans(Gv):
    return pl.pallas_call(
        _detrans_body,
        grid=(HIST // H2, NBLK // GBT),
        in_specs=[
            pl.BlockSpec((H2, GBT, DIM, CHUNK), lambda h, g: (h, g, 0, 0))
        ],
        out_specs=pl.BlockSpec(
            (H2, 8, GBT, 8, CHUNK), lambda h, g: (h, 0, g, 0, 0)
        ),
        out_shape=jax.ShapeDtypeStruct((HIST, 8, NBLK, 8, CHUNK), jnp.float32),
    )(Gv)


_PERM = tuple(k // 2 if k % 2 == 0 else 64 + k // 2 for k in range(CHUNK))


def kernel(topic_ids, W):
    table = _relayout(W.T).reshape(TPAD, DIM)
    ids = (
        (topic_ids & ~(TBLK - 1))
        | ((topic_ids & (HALF - 1)) << 1)
        | ((topic_ids >> LOGH) & 1)
    )
    perm = jnp.asarray(_PERM, jnp.int32)
    idx = ids.T.reshape(HIST, NBLK, CHUNK)[:, :, perm].reshape(NW, PER_W, CHUNK)
    q = _gather(idx, table)
    o5 = _detrans(q.reshape(HIST, NBLK, DIM, CHUNK))
    out = o5.transpose(2, 4, 0, 1, 3).reshape(BATCH, HIST, DIM)
    return out, 0
